# baseline (speedup 1.0000x reference)
import functools

import jax
import jax.numpy as jnp
from jax.experimental import pallas as pl
from jax.experimental.pallas import tpu as pltpu


def _one_chamfer(pr_row, pi_row, tr_all, ti_all, cb, nt):
    cb = min(cb, nt)
    pn_row = pr_row * pr_row + pi_row * pi_row
    valid_row = pn_row > 1e-12
    prm = jnp.transpose(jnp.where(valid_row, pr_row, jnp.inf), (1, 0))
    pim = jnp.transpose(jnp.where(valid_row, pi_row, jnp.inf), (1, 0))
    acc = None
    t2p_sum = jnp.float32(0.0)
    for c in range(nt // cb):
        tr = tr_all[:, c * cb:(c + 1) * cb]
        ti = ti_all[:, c * cb:(c + 1) * cb]
        dr = prm - tr
        di = pim - ti
        dist = dr * dr + di * di
        t2p_sum = t2p_sum + jnp.sum(jnp.min(dist, axis=0))
        acc = dist if acc is None else jnp.minimum(acc, dist)
    rowmin = jnp.min(acc, axis=1, keepdims=True)
    rowmin_row = jnp.transpose(rowmin, (1, 0))
    p2t_sum = jnp.sum(jnp.where(valid_row, rowmin_row, 0.0))
    cnt = jnp.sum(valid_row.astype(jnp.float32))
    return p2t_sum / jnp.maximum(cnt, 1.0) + t2p_sum / nt


def _both_kernel(stacked, out, total, *, cb, ntz, nb):
    i = pl.program_id(0)
    a = stacked[0]
    pole = _one_chamfer(a[0:1], a[1:2], a[2:3], a[3:4], cb, a.shape[1])
    zero = _one_chamfer(a[4:5], a[5:6], a[6:7, :ntz], a[7:8, :ntz], cb, ntz)
    step = (pole + 0.5 * zero) * (1.0 / nb)

    @pl.when(i == 0)
    def _init():
        total[0, 0] = step

    @pl.when(i != 0)
    def _acc():
        total[0, 0] = total[0, 0] + step

    @pl.when(i == nb - 1)
    def _final():
        out[...] = jnp.reshape(total[0, 0], (1, 1))


def kernel(pred_poles, pred_zeros, target_poles_list, target_zeros_list):
    b, np_, _ = pred_poles.shape
    ntz = target_zeros_list.shape[1]

    pp = jnp.transpose(pred_poles, (0, 2, 1))
    tp = jnp.transpose(target_poles_list, (0, 2, 1))
    zp = jnp.transpose(pred_zeros, (0, 2, 1))
    tz = jnp.transpose(target_zeros_list, (0, 2, 1))
    tz = jnp.pad(tz, ((0, 0), (0, 0), (0, np_ - ntz)),
                 constant_values=1e30)
    stacked = jnp.concatenate([pp, tp, zp, tz], axis=1)

    total = pl.pallas_call(
        functools.partial(_both_kernel, cb=1024, ntz=ntz, nb=b),
        grid=(b,),
        in_specs=[pl.BlockSpec((1, 8, np_), lambda i: (i, 0, 0))],
        out_specs=pl.BlockSpec((1, 1), lambda i: (0, 0)),
        out_shape=jax.ShapeDtypeStruct((1, 1), jnp.float32),
        scratch_shapes=[pltpu.SMEM((1, 1), jnp.float32)],
        compiler_params=pltpu.CompilerParams(
            dimension_semantics=("arbitrary",),
        ),
    )(stacked)

    return total[0, 0]


# speedup vs baseline: 1.1157x; 1.1157x over previous
import functools

import jax
import jax.numpy as jnp
from jax.experimental import pallas as pl
from jax.experimental.pallas import tpu as pltpu


def _one_chamfer(pr_row, pi_row, tr_all, ti_all, cb, nt):
    cb = min(cb, nt)
    prm = jnp.transpose(pr_row, (1, 0))
    pim = jnp.transpose(pi_row, (1, 0))
    acc = None
    t2p_sum = jnp.float32(0.0)
    for c in range(nt // cb):
        tr = tr_all[:, c * cb:(c + 1) * cb]
        ti = ti_all[:, c * cb:(c + 1) * cb]
        dr = prm - tr
        di = pim - ti
        dist = dr * dr + di * di
        t2p_sum = t2p_sum + jnp.sum(jnp.min(dist, axis=0))
        acc = dist if acc is None else jnp.minimum(acc, dist)
    rowmin = jnp.min(acc, axis=1, keepdims=True)
    finite = rowmin < jnp.inf
    p2t_sum = jnp.sum(jnp.where(finite, rowmin, 0.0))
    cnt = jnp.sum(finite.astype(jnp.float32))
    return p2t_sum / jnp.maximum(cnt, 1.0) + t2p_sum / nt


def _both_kernel(stacked, out, total, *, cb, ntz, nb):
    i = pl.program_id(0)
    a = stacked[0]
    pole = _one_chamfer(a[0:1], a[1:2], a[2:3], a[3:4], cb, a.shape[1])
    zero = _one_chamfer(a[4:5], a[5:6], a[6:7, :ntz], a[7:8, :ntz], cb, ntz)
    step = (pole + 0.5 * zero) * (1.0 / nb)

    @pl.when(i == 0)
    def _init():
        total[0, 0] = step

    @pl.when(i != 0)
    def _acc():
        total[0, 0] = total[0, 0] + step

    @pl.when(i == nb - 1)
    def _final():
        out[...] = jnp.reshape(total[0, 0], (1, 1))


def kernel(pred_poles, pred_zeros, target_poles_list, target_zeros_list):
    b, np_, _ = pred_poles.shape
    ntz = target_zeros_list.shape[1]

    def _mask_pred(pred):
        mag2 = jnp.sum(pred * pred, axis=-1, keepdims=True)
        return jnp.where(mag2 > 1e-12, pred, jnp.inf)

    pp = jnp.transpose(_mask_pred(pred_poles), (0, 2, 1))
    tp = jnp.transpose(target_poles_list, (0, 2, 1))
    zp = jnp.transpose(_mask_pred(pred_zeros), (0, 2, 1))
    tz = jnp.transpose(target_zeros_list, (0, 2, 1))
    tz = jnp.pad(tz, ((0, 0), (0, 0), (0, np_ - ntz)),
                 constant_values=1e30)
    stacked = jnp.concatenate([pp, tp, zp, tz], axis=1)

    total = pl.pallas_call(
        functools.partial(_both_kernel, cb=1024, ntz=ntz, nb=b),
        grid=(b,),
        in_specs=[pl.BlockSpec((1, 8, np_), lambda i: (i, 0, 0))],
        out_specs=pl.BlockSpec((1, 1), lambda i: (0, 0)),
        out_shape=jax.ShapeDtypeStruct((1, 1), jnp.float32),
        scratch_shapes=[pltpu.SMEM((1, 1), jnp.float32)],
        compiler_params=pltpu.CompilerParams(
            dimension_semantics=("arbitrary",),
        ),
    )(stacked)

    return total[0, 0]


# speedup vs baseline: 1.1192x; 1.0031x over previous
import functools

import jax
import jax.numpy as jnp
from jax.experimental import pallas as pl
from jax.experimental.pallas import tpu as pltpu


def _one_chamfer(pr_row, pi_row, tr_all, ti_all, cb, nt):
    cb = min(cb, nt)
    pr = jnp.transpose(pr_row, (1, 0))
    pi = jnp.transpose(pi_row, (1, 0))
    pn = pr * pr + pi * pi
    valid = pn > 1e-12
    prm = jnp.where(valid, pr, jnp.inf)
    pim = jnp.where(valid, pi, jnp.inf)
    acc = None
    t2p_sum = jnp.float32(0.0)
    for c in range(nt // cb):
        tr = tr_all[:, c * cb:(c + 1) * cb]
        ti = ti_all[:, c * cb:(c + 1) * cb]
        dr = prm - tr
        di = pim - ti
        dist = dr * dr + di * di
        t2p_sum = t2p_sum + jnp.sum(jnp.min(dist, axis=0))
        acc = dist if acc is None else jnp.minimum(acc, dist)
    rowmin = jnp.min(acc, axis=1, keepdims=True)
    p2t_sum = jnp.sum(jnp.where(valid, rowmin, 0.0))
    cnt = jnp.sum(valid.astype(jnp.float32))
    return p2t_sum / jnp.maximum(cnt, 1.0) + t2p_sum / nt


def _both_kernel(stacked, out, total, *, cb, ntz, nb):
    i = pl.program_id(0)
    a = stacked[0]
    pole = _one_chamfer(a[0:1], a[1:2], a[2:3], a[3:4], cb, a.shape[1])
    zero = _one_chamfer(a[4:5], a[5:6], a[6:7, :ntz], a[7:8, :ntz], cb, ntz)
    step = (pole + 0.5 * zero) * (1.0 / nb)

    @pl.when(i == 0)
    def _init():
        total[0, 0] = step

    @pl.when(i != 0)
    def _acc():
        total[0, 0] = total[0, 0] + step

    @pl.when(i == nb - 1)
    def _final():
        out[...] = jnp.reshape(total[0, 0], (1, 1))


def kernel(pred_poles, pred_zeros, target_poles_list, target_zeros_list):
    b, np_, _ = pred_poles.shape
    ntz = target_zeros_list.shape[1]

    pp = jnp.transpose(pred_poles, (0, 2, 1))
    tp = jnp.transpose(target_poles_list, (0, 2, 1))
    zp = jnp.transpose(pred_zeros, (0, 2, 1))
    tz = jnp.transpose(target_zeros_list, (0, 2, 1))
    tz = jnp.pad(tz, ((0, 0), (0, 0), (0, np_ - ntz)),
                 constant_values=1e30)
    stacked = jnp.concatenate([pp, tp, zp, tz], axis=1)

    total = pl.pallas_call(
        functools.partial(_both_kernel, cb=1024, ntz=ntz, nb=b),
        grid=(b,),
        in_specs=[pl.BlockSpec((1, 8, np_), lambda i: (i, 0, 0))],
        out_specs=pl.BlockSpec((1, 1), lambda i: (0, 0)),
        out_shape=jax.ShapeDtypeStruct((1, 1), jnp.float32),
        scratch_shapes=[pltpu.SMEM((1, 1), jnp.float32)],
        compiler_params=pltpu.CompilerParams(
            dimension_semantics=("arbitrary",),
        ),
    )(stacked)

    return total[0, 0]
